# baseline (speedup 1.0000x reference)
import functools

import jax
import jax.numpy as jnp
from jax import lax
from jax.experimental import pallas as pl
from jax.experimental.pallas import tpu as pltpu
from jax.experimental.pallas import tpu_sc as plsc

VOCAB = 1000000
HIDDEN = 64
HP = 128
BATCH = 4096
SEQ = 200

N = BATCH * SEQ
NUM_WORKERS = 32
PER_WORKER = N // NUM_WORKERS
CHUNK = 256
NUM_CHUNKS = PER_WORKER // CHUNK

VBLK = 128
NBLK = VOCAB // VBLK
TAIL = VOCAB - NBLK * VBLK
PAIRS = NBLK // (2 * NUM_WORKERS)
REM0 = 2 * NUM_WORKERS * PAIRS

_mesh = plsc.VectorSubcoreMesh(core_axis_name="c", subcore_axis_name="s")


def _transpose_block(src_v, dst_v, ncols):
    for h in range(HIDDEN):
        for vg in range(ncols // 16):
            val = src_v[h, pl.ds(vg * 16, 16)]
            v_idx = jax.lax.iota(jnp.int32, 16) + (vg * 16)
            h_idx = jnp.full((16,), h, jnp.int32)
            plsc.store_scatter(dst_v, [v_idx, h_idx], val)


@functools.partial(
    pl.kernel,
    out_type=jax.ShapeDtypeStruct((VOCAB, HP), jnp.float32),
    mesh=_mesh,
    scratch_types=[
        pltpu.VMEM((2, HIDDEN, VBLK), jnp.float32),
        pltpu.VMEM((2, VBLK, HP), jnp.float32),
        pltpu.VMEM((HIDDEN, TAIL), jnp.float32),
        pltpu.SemaphoreType.DMA,
        pltpu.SemaphoreType.DMA,
        pltpu.SemaphoreType.DMA,
        pltpu.SemaphoreType.DMA,
    ],
    compiler_params=pltpu.CompilerParams(needs_layout_passes=False),
)
def _repack_kernel(tt_hbm, tp_hbm, src_v, dst_v, tail_v, sr0, sr1, sw0, sw1):
    wid = lax.axis_index("s") * 2 + lax.axis_index("c")
    sr = (sr0, sr1)
    sw = (sw0, sw1)

    def blk_base(j, b):
        return (wid + NUM_WORKERS * (2 * j + b)) * VBLK

    def start_read(j, b):
        return pltpu.async_copy(
            tt_hbm.at[:, pl.ds(blk_base(j, b), VBLK)], src_v.at[b], sr[b]
        )

    def start_write(j, b):
        return pltpu.async_copy(
            dst_v.at[b], tp_hbm.at[pl.ds(blk_base(j, b), VBLK), :], sw[b]
        )

    start_read(0, 0)
    start_read(0, 1)

    def wait_read(b):
        pltpu.make_async_copy(
            tt_hbm.at[:, pl.ds(0, VBLK)], src_v.at[b], sr[b]
        ).wait()

    def wait_write(b):
        pltpu.make_async_copy(
            dst_v.at[b], tp_hbm.at[pl.ds(0, VBLK), :], sw[b]
        ).wait()

    def body(j, carry):
        for b in (0, 1):
            @pl.when(j > 0)
            def _():
                wait_write(b)

            wait_read(b)
            _transpose_block(src_v.at[b], dst_v.at[b], VBLK)
            start_write(j, b)

            @pl.when(j + 1 < PAIRS)
            def _():
                start_read(j + 1, b)

        return carry

    lax.fori_loop(0, PAIRS, body, 0)
    for b in (0, 1):
        wait_write(b)

    @pl.when(wid < NBLK - REM0)
    def _():
        v0 = (REM0 + wid) * VBLK
        pltpu.sync_copy(tt_hbm.at[:, pl.ds(v0, VBLK)], src_v.at[0])
        _transpose_block(src_v.at[0], dst_v.at[0], VBLK)
        pltpu.sync_copy(dst_v.at[0], tp_hbm.at[pl.ds(v0, VBLK), :])

    @pl.when(wid == NUM_WORKERS - 1)
    def _():
        v0 = NBLK * VBLK
        pltpu.sync_copy(tt_hbm.at[:, pl.ds(v0, TAIL)], tail_v)
        _transpose_block(tail_v, dst_v.at[1], TAIL)
        pltpu.sync_copy(
            dst_v.at[1].at[pl.ds(0, TAIL), :],
            tp_hbm.at[pl.ds(v0, TAIL), :],
        )


@functools.partial(
    pl.kernel,
    out_type=jax.ShapeDtypeStruct((N, HP), jnp.float32),
    mesh=_mesh,
    scratch_types=[
        pltpu.VMEM((PER_WORKER,), jnp.int32),
        pltpu.VMEM((2, CHUNK, HP), jnp.float32),
        pltpu.SemaphoreType.DMA,
        pltpu.SemaphoreType.DMA,
        pltpu.SemaphoreType.DMA,
        pltpu.SemaphoreType.DMA,
        pltpu.SemaphoreType.DMA,
    ],
)
def _gather_kernel(idx_hbm, table_hbm, out_hbm, idx_v, rows_v, sg0, sg1, sw0, sw1, si):
    wid = lax.axis_index("s") * 2 + lax.axis_index("c")
    s_w = (25 * wid) // 4
    b_w = 1024 * (wid % 4)
    base = wid * PER_WORKER

    def chunk_pos(c):
        t = b_w + c * CHUNK
        s_extra = t // BATCH
        return s_w + s_extra, t - s_extra * BATCH

    idx_descs = []
    for c in range(NUM_CHUNKS):
        s_c, b_c = chunk_pos(c)
        idx_descs.append(
            pltpu.async_copy(
                idx_hbm.at[s_c, pl.ds(b_c, CHUNK)],
                idx_v.at[pl.ds(c * CHUNK, CHUNK)],
                si,
            )
        )
    for d in idx_descs:
        d.wait()

    sg = (sg0, sg1)
    sw = (sw0, sw1)

    def start_gather(c, b):
        return pltpu.async_copy(
            table_hbm.at[idx_v.at[pl.ds(c * CHUNK, CHUNK)]],
            rows_v.at[b],
            sg[b],
        )

    def start_write(c, b):
        return pltpu.async_copy(
            rows_v.at[b],
            out_hbm.at[pl.ds(base + c * CHUNK, CHUNK), :],
            sw[b],
        )

    gather_d = [None] * NUM_CHUNKS
    write_d = [None] * NUM_CHUNKS
    gather_d[0] = start_gather(0, 0)
    for i in range(NUM_CHUNKS):
        b = i & 1
        if i + 1 < NUM_CHUNKS:
            if i >= 1:
                write_d[i - 1].wait()
            gather_d[i + 1] = start_gather(i + 1, 1 - b)
        gather_d[i].wait()
        write_d[i] = start_write(i, b)
    write_d[NUM_CHUNKS - 2].wait()
    write_d[NUM_CHUNKS - 1].wait()


def kernel(input_ids, word_embedding):
    table_p = _repack_kernel(word_embedding.T)
    out = _gather_kernel(input_ids.T, table_p)
    loss = jnp.zeros((), dtype=jnp.float32)
    out = out[:, :HIDDEN].reshape(SEQ, BATCH, HIDDEN).transpose(1, 0, 2)
    return (loss, out)


# speedup vs baseline: 1.7277x; 1.7277x over previous
import functools

import jax
import jax.numpy as jnp
from jax import lax
from jax.experimental import pallas as pl
from jax.experimental.pallas import tpu as pltpu
from jax.experimental.pallas import tpu_sc as plsc

VOCAB = 1000000
HIDDEN = 64
HP = 128
BATCH = 4096
SEQ = 200

N = BATCH * SEQ
NUM_WORKERS = 32
PER_WORKER = N // NUM_WORKERS
CHUNK = 256
NUM_CHUNKS = PER_WORKER // CHUNK
NBUF = 3

_mesh = plsc.VectorSubcoreMesh(core_axis_name="c", subcore_axis_name="s")


@functools.partial(
    pl.kernel,
    out_type=jax.ShapeDtypeStruct((N, HP), jnp.float32),
    mesh=_mesh,
    scratch_types=[
        pltpu.VMEM((PER_WORKER,), jnp.int32),
        pltpu.VMEM((NBUF, CHUNK, HP), jnp.float32),
        pltpu.SemaphoreType.DMA,
        pltpu.SemaphoreType.DMA,
        pltpu.SemaphoreType.DMA,
        pltpu.SemaphoreType.DMA,
        pltpu.SemaphoreType.DMA,
        pltpu.SemaphoreType.DMA,
        pltpu.SemaphoreType.DMA,
    ],
)
def _gather_kernel(
    idx_hbm, table_hbm, out_hbm, idx_v, rows_v, sg0, sg1, sg2, sw0, sw1, sw2, si
):
    wid = lax.axis_index("s") * 2 + lax.axis_index("c")
    s_w = (25 * wid) // 4
    b_w = 1024 * (wid % 4)
    base = wid * PER_WORKER

    def chunk_pos(c):
        t = b_w + c * CHUNK
        s_extra = t // BATCH
        return s_w + s_extra, t - s_extra * BATCH

    idx_descs = []
    for c in range(NUM_CHUNKS):
        s_c, b_c = chunk_pos(c)
        idx_descs.append(
            pltpu.async_copy(
                idx_hbm.at[s_c, pl.ds(b_c, CHUNK)],
                idx_v.at[pl.ds(c * CHUNK, CHUNK)],
                si,
            )
        )
    for d in idx_descs:
        d.wait()

    sg = (sg0, sg1, sg2)
    sw = (sw0, sw1, sw2)

    def start_gather(c, b):
        return pltpu.async_copy(
            table_hbm.at[idx_v.at[pl.ds(c * CHUNK, CHUNK)]],
            rows_v.at[b],
            sg[b],
        )

    def start_write(c, b):
        return pltpu.async_copy(
            rows_v.at[b],
            out_hbm.at[pl.ds(base + c * CHUNK, CHUNK), :],
            sw[b],
        )

    gather_d = [None] * NUM_CHUNKS
    write_d = [None] * NUM_CHUNKS
    for c in range(NBUF - 1):
        gather_d[c] = start_gather(c, c)
    for i in range(NUM_CHUNKS):
        b = i % NBUF
        if i + NBUF - 1 < NUM_CHUNKS:
            nb = (i + NBUF - 1) % NBUF
            if i >= 1:
                write_d[i - 1].wait()
            gather_d[i + NBUF - 1] = start_gather(i + NBUF - 1, nb)
        gather_d[i].wait()
        write_d[i] = start_write(i, b)
    for i in range(max(0, NUM_CHUNKS - NBUF + 1), NUM_CHUNKS):
        write_d[i].wait()


def kernel(input_ids, word_embedding):
    table_p = jnp.pad(word_embedding, ((0, 0), (0, HP - HIDDEN)))
    out = _gather_kernel(input_ids.T, table_p)
    loss = jnp.zeros((), dtype=jnp.float32)
    out = out[:, :HIDDEN].reshape(SEQ, BATCH, HIDDEN).transpose(1, 0, 2)
    return (loss, out)


# speedup vs baseline: 1.7288x; 1.0006x over previous
import functools

import jax
import jax.numpy as jnp
from jax import lax
from jax.experimental import pallas as pl
from jax.experimental.pallas import tpu as pltpu
from jax.experimental.pallas import tpu_sc as plsc

VOCAB = 1000000
HIDDEN = 64
HP = 128
BATCH = 4096
SEQ = 200

N = BATCH * SEQ
NUM_WORKERS = 32
PER_WORKER = N // NUM_WORKERS
CHUNK = 256
NUM_CHUNKS = PER_WORKER // CHUNK
NBUF = 3

_mesh = plsc.VectorSubcoreMesh(core_axis_name="c", subcore_axis_name="s")


@functools.partial(
    pl.kernel,
    out_type=jax.ShapeDtypeStruct((N, HP), jnp.float32),
    mesh=_mesh,
    scratch_types=[
        pltpu.VMEM((PER_WORKER,), jnp.int32),
        pltpu.VMEM((NBUF, CHUNK, HP), jnp.float32),
        pltpu.SemaphoreType.DMA,
        pltpu.SemaphoreType.DMA,
        pltpu.SemaphoreType.DMA,
        pltpu.SemaphoreType.DMA,
        pltpu.SemaphoreType.DMA,
        pltpu.SemaphoreType.DMA,
        pltpu.SemaphoreType.DMA,
    ],
)
def _gather_kernel(
    idx_hbm, table_hbm, out_hbm, idx_v, rows_v, sg0, sg1, sg2, sw0, sw1, sw2, si
):
    wid = lax.axis_index("s") * 2 + lax.axis_index("c")
    s_w = (25 * wid) // 4
    b_w = 1024 * (wid % 4)
    base = wid * PER_WORKER

    def chunk_pos(c):
        t = b_w + c * CHUNK
        s_extra = t // BATCH
        return s_w + s_extra, t - s_extra * BATCH

    idx_descs = []
    for c in range(NUM_CHUNKS):
        s_c, b_c = chunk_pos(c)
        idx_descs.append(
            pltpu.async_copy(
                idx_hbm.at[s_c, pl.ds(b_c, CHUNK)],
                idx_v.at[pl.ds(c * CHUNK, CHUNK)],
                si,
            )
        )
    for d in idx_descs:
        d.wait()

    sg = (sg0, sg1, sg2)
    sw = (sw0, sw1, sw2)

    def start_gather(c, b):
        return pltpu.async_copy(
            table_hbm.at[idx_v.at[pl.ds(c * CHUNK, CHUNK)]],
            rows_v.at[b],
            sg[b],
        )

    def start_write(c, b):
        return pltpu.async_copy(
            rows_v.at[b],
            out_hbm.at[pl.ds(base + c * CHUNK, CHUNK), :],
            sw[b],
        )

    gather_d = [None] * NUM_CHUNKS
    write_d = [None] * NUM_CHUNKS
    for c in range(NBUF - 1):
        gather_d[c] = start_gather(c, c)
    for i in range(NUM_CHUNKS):
        b = i % NBUF
        if i + NBUF - 1 < NUM_CHUNKS:
            nb = (i + NBUF - 1) % NBUF
            if i >= 1:
                write_d[i - 1].wait()
            gather_d[i + NBUF - 1] = start_gather(i + NBUF - 1, nb)
        gather_d[i].wait()
        write_d[i] = start_write(i, b)
    for i in range(NUM_CHUNKS - NBUF, NUM_CHUNKS):
        write_d[i].wait()


def kernel(input_ids, word_embedding):
    table_p = jnp.pad(word_embedding, ((0, 0), (0, HP - HIDDEN)))
    out = _gather_kernel(input_ids.T, table_p)
    loss = jnp.zeros((), dtype=jnp.float32)
    out = out[:, :HIDDEN].reshape(SEQ, BATCH, HIDDEN).transpose(1, 0, 2)
    return (loss, out)
